# Initial kernel scaffold; baseline (speedup 1.0000x reference)
import jax
import jax.numpy as jnp
from jax.experimental import pallas as pl


def kernel(pred, shrink_map, shrink_mask, threshold_map, threshold_mask):
    raise NotImplementedError("write your pallas kernel here")


# speedup vs baseline: 13.1347x; 13.1347x over previous
import functools

import jax
import jax.numpy as jnp
from jax import lax
from jax.experimental import pallas as pl
from jax.experimental.pallas import tpu as pltpu
from jax.experimental.pallas import tpu_sc as plsc

ALPHA = 1.0
BETA = 10.0
OHEM_RATIO = 3.0
EPS = 1e-06

N, C, H, W = 16, 3, 512, 512
TOTAL = N * H * W

VMAX = 16.2
NB1 = 4096
NB2 = 2048
SCALE1 = NB1 / VMAX
W1 = VMAX / NB1
SCALE2 = NB2 / W1


def _pass_a_body(pred_ref, smap_ref, smask_ref, tmap_ref, tmask_ref,
                 sums_ref, negloss_ref):
    sm = pred_ref[0, 0]
    tm = pred_ref[0, 1]
    bm = pred_ref[0, 2]
    y = smap_ref[0]
    m = smask_ref[0]
    t = tmap_ref[0]
    tmk = tmask_ref[0]

    pos = y * m
    neg = (1.0 - y) * m
    p = jnp.clip(sm, 1e-7, 1.0 - 1e-7)
    bce = -jnp.log(jnp.where(y > 0.5, p, 1.0 - p))

    pos_loss = bce * pos
    neg_loss = bce * neg
    negloss_ref[0] = jnp.where(neg > 0.5, neg_loss, -1.0)

    s0 = jnp.sum(pos_loss)
    s1 = jnp.sum(pos)
    s2 = jnp.sum(neg)
    s3 = jnp.sum(neg_loss)
    s4 = jnp.sum(jnp.abs(tm - t) * tmk)
    s5 = jnp.sum(tmk)
    s6 = jnp.sum(bm * y * m)
    s7 = jnp.sum(bm * m)
    s8 = jnp.sum(y * m)

    lane = lax.broadcasted_iota(jnp.int32, (1, 1, 128), 2)
    vec = jnp.zeros((1, 1, 128), jnp.float32)
    for j, s in enumerate((s0, s1, s2, s3, s4, s5, s6, s7, s8)):
        vec = jnp.where(lane == j, s, vec)
    sums_ref[...] = vec


def _pass_a(pred, smap, smask, tmap, tmask):
    return pl.pallas_call(
        _pass_a_body,
        grid=(N,),
        in_specs=[
            pl.BlockSpec((1, C, H, W), lambda i: (i, 0, 0, 0)),
            pl.BlockSpec((1, H, W), lambda i: (i, 0, 0)),
            pl.BlockSpec((1, H, W), lambda i: (i, 0, 0)),
            pl.BlockSpec((1, H, W), lambda i: (i, 0, 0)),
            pl.BlockSpec((1, H, W), lambda i: (i, 0, 0)),
        ],
        out_specs=[
            pl.BlockSpec((1, 1, 128), lambda i: (i, 0, 0)),
            pl.BlockSpec((1, H, W), lambda i: (i, 0, 0)),
        ],
        out_shape=[
            jax.ShapeDtypeStruct((N, 1, 128), jnp.float32),
            jax.ShapeDtypeStruct((N, H, W), jnp.float32),
        ],
    )(pred, smap, smask, tmap, tmask)


NCORES = 2
NSUB = 16
NWORK = NCORES * NSUB
ELEMS = TOTAL // NWORK
CHUNK = 16384
NCH = ELEMS // CHUNK
LANES = 16

@functools.cache
def _sc_mesh():
    return plsc.VectorSubcoreMesh(core_axis_name="c", subcore_axis_name="s",
                                  num_cores=NCORES, num_subcores=NSUB)


def _stream_chunks(nl_hbm, base, bufs, sems, process_chunk, carry):
    handles = [None, None]
    handles[0] = pltpu.async_copy(
        nl_hbm.at[pl.ds(base, CHUNK)], bufs[0], sems[0])
    for c in range(NCH):
        if c + 1 < NCH:
            handles[(c + 1) % 2] = pltpu.async_copy(
                nl_hbm.at[pl.ds(base + (c + 1) * CHUNK, CHUNK)],
                bufs[(c + 1) % 2], sems[(c + 1) % 2])
        handles[c % 2].wait()
        carry = process_chunk(bufs[c % 2], carry)
    return carry


def _zero_vmem_i32(ref, n):
    z = jnp.zeros((LANES,), jnp.int32)

    def body(j, _):
        ref[pl.ds(j * LANES, LANES)] = z
        return 0
    lax.fori_loop(0, n // LANES, body, 0)


def _zero_vmem_f32(ref, n):
    z = jnp.zeros((LANES,), jnp.float32)

    def body(j, _):
        ref[pl.ds(j * LANES, LANES)] = z
        return 0
    lax.fori_loop(0, n // LANES, body, 0)


def _lane_reduce_i32(hist, red, nbins):
    def body(j, _):
        acc = jnp.zeros((LANES,), jnp.int32)
        for l in range(LANES):
            acc = acc + hist[pl.ds(l * nbins + j * LANES, LANES)]
        red[pl.ds(j * LANES, LANES)] = acc
        return 0
    lax.fori_loop(0, nbins // LANES, body, 0)


def _lane_reduce_f32(hist, red, nbins):
    def body(j, _):
        acc = jnp.zeros((LANES,), jnp.float32)
        for l in range(LANES):
            acc = acc + hist[pl.ds(l * nbins + j * LANES, LANES)]
        red[pl.ds(j * LANES, LANES)] = acc
        return 0
    lax.fori_loop(0, nbins // LANES, body, 0)


def _hist1_sc_body(nl_hbm, out_hbm, buf0, buf1, hist, red, sem0, sem1):
    wid = lax.axis_index("s") * NCORES + lax.axis_index("c")
    base = wid * ELEMS
    lane = lax.iota(jnp.int32, LANES)
    ones = jnp.ones((LANES,), jnp.int32)

    _zero_vmem_i32(hist, LANES * NB1)

    def process_chunk(buf, carry):
        def body(i, _):
            v = buf[pl.ds(i * LANES, LANES)]
            idx = jnp.minimum(jnp.maximum(
                (v * SCALE1).astype(jnp.int32), 0), NB1 - 1)
            plsc.addupdate_scatter(hist, [lane * NB1 + idx], ones,
                                   mask=v >= 0.0)
            return 0
        lax.fori_loop(0, CHUNK // LANES, body, 0)
        return carry

    _stream_chunks(nl_hbm, base, (buf0, buf1), (sem0, sem1),
                   process_chunk, 0)
    _lane_reduce_i32(hist, red, NB1)
    pltpu.sync_copy(red, out_hbm.at[wid])


@functools.cache
def _hist1_sc():
    return pl.kernel(
        _hist1_sc_body,
        out_type=jax.ShapeDtypeStruct((NWORK, NB1), jnp.int32),
        mesh=_sc_mesh(),
        compiler_params=pltpu.CompilerParams(needs_layout_passes=False),
        scratch_types=[
            pltpu.VMEM((CHUNK,), jnp.float32),
            pltpu.VMEM((CHUNK,), jnp.float32),
            pltpu.VMEM((LANES * NB1,), jnp.int32),
            pltpu.VMEM((NB1,), jnp.int32),
            pltpu.SemaphoreType.DMA,
            pltpu.SemaphoreType.DMA,
        ],
    )


def _hist2_sc_body(nl_hbm, params_hbm, ocnt_hbm, osum_hbm, oabove_hbm,
                   buf0, buf1, pbuf, hcnt, hsum, rcnt, rsum, avec,
                   sem0, sem1):
    wid = lax.axis_index("s") * NCORES + lax.axis_index("c")
    base = wid * ELEMS
    lane = lax.iota(jnp.int32, LANES)
    ones = jnp.ones((LANES,), jnp.int32)

    pltpu.sync_copy(params_hbm, pbuf)
    t1i = pbuf[0].astype(jnp.int32)
    lov = pbuf[1]

    _zero_vmem_i32(hcnt, LANES * NB2)
    _zero_vmem_f32(hsum, LANES * NB2)

    def process_chunk(buf, acc):
        def body(i, acc):
            v = buf[pl.ds(i * LANES, LANES)]
            valid = v >= 0.0
            b1 = jnp.minimum(jnp.maximum(
                (v * SCALE1).astype(jnp.int32), 0), NB1 - 1)
            gt = valid & (b1 > t1i)
            eq = valid & (b1 == t1i)
            acc = acc + jnp.where(gt, v, 0.0)
            b2 = jnp.minimum(jnp.maximum(
                ((v - lov) * SCALE2).astype(jnp.int32), 0), NB2 - 1)
            addr = lane * NB2 + b2
            plsc.addupdate_scatter(hcnt, [addr], ones, mask=eq)
            plsc.addupdate_scatter(hsum, [addr], v, mask=eq)
            return acc
        return lax.fori_loop(0, CHUNK // LANES, body, acc)

    acc = _stream_chunks(nl_hbm, base, (buf0, buf1), (sem0, sem1),
                         process_chunk, jnp.zeros((LANES,), jnp.float32))
    avec[...] = acc
    _lane_reduce_i32(hcnt, rcnt, NB2)
    _lane_reduce_f32(hsum, rsum, NB2)
    pltpu.sync_copy(rcnt, ocnt_hbm.at[wid])
    pltpu.sync_copy(rsum, osum_hbm.at[wid])
    pltpu.sync_copy(avec, oabove_hbm.at[wid])


@functools.cache
def _hist2_sc():
    return pl.kernel(
        _hist2_sc_body,
        out_type=(
            jax.ShapeDtypeStruct((NWORK, NB2), jnp.int32),
            jax.ShapeDtypeStruct((NWORK, NB2), jnp.float32),
            jax.ShapeDtypeStruct((NWORK, LANES), jnp.float32),
        ),
        mesh=_sc_mesh(),
        compiler_params=pltpu.CompilerParams(needs_layout_passes=False),
        scratch_types=[
            pltpu.VMEM((CHUNK,), jnp.float32),
            pltpu.VMEM((CHUNK,), jnp.float32),
            pltpu.VMEM((8, LANES), jnp.float32),
            pltpu.VMEM((LANES * NB2,), jnp.int32),
            pltpu.VMEM((LANES * NB2,), jnp.float32),
            pltpu.VMEM((NB2,), jnp.int32),
            pltpu.VMEM((NB2,), jnp.float32),
            pltpu.VMEM((LANES,), jnp.float32),
            pltpu.SemaphoreType.DMA,
            pltpu.SemaphoreType.DMA,
        ],
    )


def kernel(pred, shrink_map, shrink_mask, threshold_map, threshold_mask):
    sums, negloss = _pass_a(pred, shrink_map, shrink_mask,
                            threshold_map, threshold_mask)
    s = jnp.sum(sums[:, 0, :9], axis=0)
    pos_loss_sum, pos_count, neg_count = s[0], s[1], s[2]
    neg_sum_total, l1_num, l1_den = s[3], s[4], s[5]
    inter, dice_a, dice_b = s[6], s[7], s[8]

    k = jnp.minimum(neg_count, pos_count * OHEM_RATIO)

    nl = negloss.reshape(-1)

    cnt1 = _hist1_sc()(nl).sum(axis=0)
    ssum1 = jnp.concatenate(
        [jnp.cumsum(cnt1[::-1])[::-1].astype(jnp.float32), jnp.zeros((1,), jnp.float32)])
    bins1 = jnp.arange(NB1)
    t1 = jnp.max(jnp.where(ssum1[:NB1] > k, bins1, -1))
    c_above1 = ssum1[t1 + 1]
    r1 = k - c_above1
    lo = t1.astype(jnp.float32) * W1

    params = jnp.stack([
        jnp.full((LANES,), t1.astype(jnp.float32)),
        jnp.full((LANES,), lo),
    ] + [jnp.zeros((LANES,), jnp.float32)] * 6)
    cnt2w, sum2w, abovew = _hist2_sc()(nl, params)
    cnt2 = cnt2w.sum(axis=0)
    sum2 = sum2w.sum(axis=0)
    sum_above1 = abovew.sum()
    ssum2 = jnp.concatenate(
        [jnp.cumsum(cnt2[::-1])[::-1].astype(jnp.float32), jnp.zeros((1,), jnp.float32)])
    vsum2 = jnp.concatenate(
        [jnp.cumsum(sum2[::-1])[::-1], jnp.zeros((1,), jnp.float32)])
    bins2 = jnp.arange(NB2)
    t2 = jnp.max(jnp.where(ssum2[:NB2] > r1, bins2, -1))
    c_above2 = ssum2[t2 + 1]
    sum_above2 = vsum2[t2 + 1]
    r2 = r1 - c_above2
    avg_t2 = jnp.where(t2 >= 0,
                       sum2[t2] / jnp.maximum(cnt2[t2].astype(jnp.float32), 1.0),
                       0.0)
    s_sel = sum_above1 + sum_above2 + r2 * avg_t2

    topk_sum = jnp.where(k >= neg_count, neg_sum_total, s_sel)

    loss_shrink = (pos_loss_sum + topk_sum) / (pos_count + k + EPS)
    loss_threshold = l1_num / (l1_den + EPS)
    loss_binary = 1.0 - 2.0 * inter / (dice_a + dice_b + EPS)
    loss_all = ALPHA * loss_shrink + BETA * loss_threshold + loss_binary
    return (loss_all, loss_shrink, loss_threshold, loss_binary)


# speedup vs baseline: 15.0266x; 1.1440x over previous
import functools

import jax
import jax.numpy as jnp
from jax import lax
from jax.experimental import pallas as pl
from jax.experimental.pallas import tpu as pltpu
from jax.experimental.pallas import tpu_sc as plsc

ALPHA = 1.0
BETA = 10.0
OHEM_RATIO = 3.0
EPS = 1e-06

N, C, H, W = 16, 3, 512, 512
TOTAL = N * H * W

VMAX = 16.2
NB1 = 4096
NB2 = 4096
SCALE1 = NB1 / VMAX
W1 = VMAX / NB1
SCALE2 = NB2 / W1


def _pass_a_body(pred_ref, smap_ref, smask_ref, tmap_ref, tmask_ref,
                 sums_ref, negloss_ref):
    sm = pred_ref[0, 0]
    tm = pred_ref[0, 1]
    bm = pred_ref[0, 2]
    y = smap_ref[0]
    m = smask_ref[0]
    t = tmap_ref[0]
    tmk = tmask_ref[0]

    pos = y * m
    neg = (1.0 - y) * m
    p = jnp.clip(sm, 1e-7, 1.0 - 1e-7)
    bce = -jnp.log(jnp.where(y > 0.5, p, 1.0 - p))

    pos_loss = bce * pos
    neg_loss = bce * neg
    negloss_ref[0] = jnp.where(neg > 0.5, neg_loss, 17.0)

    s0 = jnp.sum(pos_loss)
    s1 = jnp.sum(pos)
    s2 = jnp.sum(neg)
    s3 = jnp.sum(neg_loss)
    s4 = jnp.sum(jnp.abs(tm - t) * tmk)
    s5 = jnp.sum(tmk)
    s6 = jnp.sum(bm * y * m)
    s7 = jnp.sum(bm * m)
    s8 = jnp.sum(y * m)

    lane = lax.broadcasted_iota(jnp.int32, (1, 1, 128), 2)
    vec = jnp.zeros((1, 1, 128), jnp.float32)
    for j, s in enumerate((s0, s1, s2, s3, s4, s5, s6, s7, s8)):
        vec = jnp.where(lane == j, s, vec)
    sums_ref[...] = vec


def _pass_a(pred, smap, smask, tmap, tmask):
    return pl.pallas_call(
        _pass_a_body,
        grid=(N,),
        in_specs=[
            pl.BlockSpec((1, C, H, W), lambda i: (i, 0, 0, 0)),
            pl.BlockSpec((1, H, W), lambda i: (i, 0, 0)),
            pl.BlockSpec((1, H, W), lambda i: (i, 0, 0)),
            pl.BlockSpec((1, H, W), lambda i: (i, 0, 0)),
            pl.BlockSpec((1, H, W), lambda i: (i, 0, 0)),
        ],
        out_specs=[
            pl.BlockSpec((1, 1, 128), lambda i: (i, 0, 0)),
            pl.BlockSpec((1, H, W), lambda i: (i, 0, 0)),
        ],
        out_shape=[
            jax.ShapeDtypeStruct((N, 1, 128), jnp.float32),
            jax.ShapeDtypeStruct((N, H, W), jnp.float32),
        ],
    )(pred, smap, smask, tmap, tmask)


NCORES = 2
NSUB = 16
NWORK = NCORES * NSUB
ELEMS = TOTAL // NWORK
CHUNK = 16384
NCH = ELEMS // CHUNK
LANES = 16
UN = 8
STRIDE1 = NB1 + 1
STRIDE2 = NB2 + 1
HSZ1 = -(-LANES * STRIDE1 // (LANES * 16)) * (LANES * 16)
HSZ2 = -(-LANES * STRIDE2 // (LANES * 16)) * (LANES * 16)

@functools.cache
def _sc_mesh():
    return plsc.VectorSubcoreMesh(core_axis_name="c", subcore_axis_name="s",
                                  num_cores=NCORES, num_subcores=NSUB)


def _stream_chunks(nl_hbm, base, bufs, sems, process_chunk, carry):
    handles = [None, None]
    handles[0] = pltpu.async_copy(
        nl_hbm.at[pl.ds(base, CHUNK)], bufs[0], sems[0])
    for c in range(NCH):
        if c + 1 < NCH:
            handles[(c + 1) % 2] = pltpu.async_copy(
                nl_hbm.at[pl.ds(base + (c + 1) * CHUNK, CHUNK)],
                bufs[(c + 1) % 2], sems[(c + 1) % 2])
        handles[c % 2].wait()
        carry = process_chunk(bufs[c % 2], carry)
    return carry


def _zero_vmem_i32(ref, n):
    z = jnp.zeros((LANES,), jnp.int32)

    def body(j, _):
        for u in range(16):
            ref[pl.ds((j * 16 + u) * LANES, LANES)] = z
        return 0
    lax.fori_loop(0, n // (16 * LANES), body, 0)


def _lane_reduce_i32(hist, red, nbins, stride):
    def body(j, _):
        acc = jnp.zeros((LANES,), jnp.int32)
        for l in range(LANES):
            acc = acc + hist[pl.ds(l * stride + j * LANES, LANES)]
        red[pl.ds(j * LANES, LANES)] = acc
        return 0
    lax.fori_loop(0, nbins // LANES, body, 0)


def _hist1_sc_body(nl_hbm, out_hbm, buf0, buf1, hist, red, sem0, sem1):
    wid = lax.axis_index("s") * NCORES + lax.axis_index("c")
    base = wid * ELEMS
    lane_off = lax.iota(jnp.int32, LANES) * STRIDE1
    ones = jnp.ones((LANES,), jnp.int32)

    _zero_vmem_i32(hist, HSZ1)

    def process_chunk(buf, carry):
        def body(i, _):
            for u in range(UN):
                v = buf[pl.ds(i * (LANES * UN) + u * LANES, LANES)]
                idx = jnp.minimum((v * SCALE1).astype(jnp.int32), NB1)
                plsc.addupdate_scatter(hist, [lane_off + idx], ones)
            return 0
        lax.fori_loop(0, CHUNK // (LANES * UN), body, 0)
        return carry

    _stream_chunks(nl_hbm, base, (buf0, buf1), (sem0, sem1),
                   process_chunk, 0)
    _lane_reduce_i32(hist, red, NB1, STRIDE1)
    pltpu.sync_copy(red, out_hbm.at[wid])


@functools.cache
def _hist1_sc():
    return pl.kernel(
        _hist1_sc_body,
        out_type=jax.ShapeDtypeStruct((NWORK, NB1), jnp.int32),
        mesh=_sc_mesh(),
        compiler_params=pltpu.CompilerParams(needs_layout_passes=False),
        scratch_types=[
            pltpu.VMEM((CHUNK,), jnp.float32),
            pltpu.VMEM((CHUNK,), jnp.float32),
            pltpu.VMEM((HSZ1,), jnp.int32),
            pltpu.VMEM((NB1,), jnp.int32),
            pltpu.SemaphoreType.DMA,
            pltpu.SemaphoreType.DMA,
        ],
    )


def _hist2_sc_body(nl_hbm, params_hbm, ocnt_hbm, oabove_hbm,
                   buf0, buf1, pbuf, hcnt, rcnt, avec, sem0, sem1):
    wid = lax.axis_index("s") * NCORES + lax.axis_index("c")
    base = wid * ELEMS
    lane_off = lax.iota(jnp.int32, LANES) * STRIDE2
    ones = jnp.ones((LANES,), jnp.int32)
    zf = jnp.zeros((LANES,), jnp.float32)

    pltpu.sync_copy(params_hbm, pbuf)
    t1i = pbuf[0].astype(jnp.int32)
    lov = pbuf[1]

    _zero_vmem_i32(hcnt, HSZ2)

    def process_chunk(buf, accs):
        def body(i, accs):
            accs = list(accs)
            for u in range(UN):
                v = buf[pl.ds(i * (LANES * UN) + u * LANES, LANES)]
                b1 = jnp.minimum((v * SCALE1).astype(jnp.int32), NB1)
                gt = (b1 > t1i) & (b1 < NB1)
                eq = b1 == t1i
                accs[u] = accs[u] + jnp.where(gt, v, 0.0)
                b2 = jnp.minimum(jnp.maximum(
                    ((v - lov) * SCALE2).astype(jnp.int32), 0), NB2 - 1)
                plsc.addupdate_scatter(hcnt, [lane_off + b2], ones, mask=eq)
            return tuple(accs)
        return lax.fori_loop(0, CHUNK // (LANES * UN), body, accs)

    accs = _stream_chunks(nl_hbm, base, (buf0, buf1), (sem0, sem1),
                          process_chunk, (zf,) * UN)
    total = accs[0]
    for u in range(1, UN):
        total = total + accs[u]
    avec[...] = total
    _lane_reduce_i32(hcnt, rcnt, NB2, STRIDE2)
    pltpu.sync_copy(rcnt, ocnt_hbm.at[wid])
    pltpu.sync_copy(avec, oabove_hbm.at[wid])


@functools.cache
def _hist2_sc():
    return pl.kernel(
        _hist2_sc_body,
        out_type=(
            jax.ShapeDtypeStruct((NWORK, NB2), jnp.int32),
            jax.ShapeDtypeStruct((NWORK, LANES), jnp.float32),
        ),
        mesh=_sc_mesh(),
        compiler_params=pltpu.CompilerParams(needs_layout_passes=False),
        scratch_types=[
            pltpu.VMEM((CHUNK,), jnp.float32),
            pltpu.VMEM((CHUNK,), jnp.float32),
            pltpu.VMEM((8, LANES), jnp.float32),
            pltpu.VMEM((HSZ2,), jnp.int32),
            pltpu.VMEM((NB2,), jnp.int32),
            pltpu.VMEM((LANES,), jnp.float32),
            pltpu.SemaphoreType.DMA,
            pltpu.SemaphoreType.DMA,
        ],
    )


def kernel(pred, shrink_map, shrink_mask, threshold_map, threshold_mask):
    sums, negloss = _pass_a(pred, shrink_map, shrink_mask,
                            threshold_map, threshold_mask)
    s = jnp.sum(sums[:, 0, :9], axis=0)
    pos_loss_sum, pos_count, neg_count = s[0], s[1], s[2]
    neg_sum_total, l1_num, l1_den = s[3], s[4], s[5]
    inter, dice_a, dice_b = s[6], s[7], s[8]

    k = jnp.minimum(neg_count, pos_count * OHEM_RATIO)

    nl = negloss.reshape(-1)

    cnt1 = _hist1_sc()(nl).sum(axis=0)
    ssum1 = jnp.concatenate(
        [jnp.cumsum(cnt1[::-1])[::-1].astype(jnp.float32), jnp.zeros((1,), jnp.float32)])
    bins1 = jnp.arange(NB1)
    t1 = jnp.max(jnp.where(ssum1[:NB1] > k, bins1, -1))
    c_above1 = ssum1[t1 + 1]
    r1 = k - c_above1
    lo = t1.astype(jnp.float32) * W1

    params = jnp.stack([
        jnp.full((LANES,), t1.astype(jnp.float32)),
        jnp.full((LANES,), lo),
    ] + [jnp.zeros((LANES,), jnp.float32)] * 6)
    cnt2w, abovew = _hist2_sc()(nl, params)
    cnt2 = cnt2w.sum(axis=0)
    sum_above1 = abovew.sum()
    mids = lo + (jnp.arange(NB2, dtype=jnp.float32) + 0.5) * (W1 / NB2)
    cnt2f = cnt2.astype(jnp.float32)
    ssum2 = jnp.concatenate(
        [jnp.cumsum(cnt2f[::-1])[::-1], jnp.zeros((1,), jnp.float32)])
    vsum2 = jnp.concatenate(
        [jnp.cumsum((cnt2f * mids)[::-1])[::-1], jnp.zeros((1,), jnp.float32)])
    bins2 = jnp.arange(NB2)
    t2 = jnp.max(jnp.where(ssum2[:NB2] > r1, bins2, -1))
    c_above2 = ssum2[t2 + 1]
    sum_above2 = vsum2[t2 + 1]
    r2 = r1 - c_above2
    avg_t2 = jnp.where(t2 >= 0, mids[t2], 0.0)
    s_sel = sum_above1 + sum_above2 + r2 * avg_t2

    topk_sum = jnp.where(k >= neg_count, neg_sum_total, s_sel)

    loss_shrink = (pos_loss_sum + topk_sum) / (pos_count + k + EPS)
    loss_threshold = l1_num / (l1_den + EPS)
    loss_binary = 1.0 - 2.0 * inter / (dice_a + dice_b + EPS)
    loss_all = ALPHA * loss_shrink + BETA * loss_threshold + loss_binary
    return (loss_all, loss_shrink, loss_threshold, loss_binary)


# speedup vs baseline: 30.0251x; 1.9981x over previous
import functools

import jax
import jax.numpy as jnp
from jax import lax
from jax.experimental import pallas as pl
from jax.experimental.pallas import tpu as pltpu
from jax.experimental.pallas import tpu_sc as plsc

ALPHA = 1.0
BETA = 10.0
OHEM_RATIO = 3.0
EPS = 1e-06

N, C, H, W = 16, 3, 512, 512
TOTAL = N * H * W

VMAX = 16.2
NB1 = 4096
NB2 = 4096
SCALE1 = NB1 / VMAX
W1 = VMAX / NB1
SCALE2 = NB2 / W1
SENTINEL = 16.202


def _pass_a_body(pred_ref, smap_ref, smask_ref, tmap_ref, tmask_ref,
                 sums_ref, negloss_ref):
    sm = pred_ref[0, 0]
    tm = pred_ref[0, 1]
    bm = pred_ref[0, 2]
    y = smap_ref[0]
    m = smask_ref[0]
    t = tmap_ref[0]
    tmk = tmask_ref[0]

    pos = y * m
    neg = (1.0 - y) * m
    p = jnp.clip(sm, 1e-7, 1.0 - 1e-7)
    bce = -jnp.log(jnp.where(y > 0.5, p, 1.0 - p))

    pos_loss = bce * pos
    neg_loss = bce * neg
    negloss_ref[0] = jnp.where(neg > 0.5, neg_loss, SENTINEL)

    s0 = jnp.sum(pos_loss)
    s1 = jnp.sum(pos)
    s2 = jnp.sum(neg)
    s3 = jnp.sum(neg_loss)
    s4 = jnp.sum(jnp.abs(tm - t) * tmk)
    s5 = jnp.sum(tmk)
    s6 = jnp.sum(bm * y * m)
    s7 = jnp.sum(bm * m)
    s8 = jnp.sum(y * m)

    lane = lax.broadcasted_iota(jnp.int32, (1, 1, 128), 2)
    vec = jnp.zeros((1, 1, 128), jnp.float32)
    for j, s in enumerate((s0, s1, s2, s3, s4, s5, s6, s7, s8)):
        vec = jnp.where(lane == j, s, vec)
    sums_ref[...] = vec


def _pass_a(pred, smap, smask, tmap, tmask):
    return pl.pallas_call(
        _pass_a_body,
        grid=(N,),
        in_specs=[
            pl.BlockSpec((1, C, H, W), lambda i: (i, 0, 0, 0)),
            pl.BlockSpec((1, H, W), lambda i: (i, 0, 0)),
            pl.BlockSpec((1, H, W), lambda i: (i, 0, 0)),
            pl.BlockSpec((1, H, W), lambda i: (i, 0, 0)),
            pl.BlockSpec((1, H, W), lambda i: (i, 0, 0)),
        ],
        out_specs=[
            pl.BlockSpec((1, 1, 128), lambda i: (i, 0, 0)),
            pl.BlockSpec((1, H, W), lambda i: (i, 0, 0)),
        ],
        out_shape=[
            jax.ShapeDtypeStruct((N, 1, 128), jnp.float32),
            jax.ShapeDtypeStruct((N, H, W), jnp.float32),
        ],
    )(pred, smap, smask, tmap, tmask)


NCORES = 2
NSUB = 16
NWORK = NCORES * NSUB
ELEMS = TOTAL // NWORK
CHUNK = 16384
NCH = ELEMS // CHUNK
LANES = 16
UN = 8
STRIDE1 = NB1 + 1
STRIDE2 = NB2 + 1
HSZ1 = -(-LANES * STRIDE1 // (LANES * 16)) * (LANES * 16)
HSZ2 = -(-LANES * STRIDE2 // (LANES * 16)) * (LANES * 16)

@functools.cache
def _sc_mesh():
    return plsc.VectorSubcoreMesh(core_axis_name="c", subcore_axis_name="s",
                                  num_cores=NCORES, num_subcores=NSUB)


def _stream_chunks(nl_hbm, base, bufs, sems, process_chunk, carry):
    handles = [None, None]
    handles[0] = pltpu.async_copy(
        nl_hbm.at[pl.ds(base, CHUNK)], bufs[0], sems[0])
    for c in range(NCH):
        if c + 1 < NCH:
            handles[(c + 1) % 2] = pltpu.async_copy(
                nl_hbm.at[pl.ds(base + (c + 1) * CHUNK, CHUNK)],
                bufs[(c + 1) % 2], sems[(c + 1) % 2])
        handles[c % 2].wait()
        carry = process_chunk(bufs[c % 2], carry)
    return carry


def _zero_vmem_i32(ref, n):
    z = jnp.zeros((LANES,), jnp.int32)

    def body(j, _):
        for u in range(16):
            ref[pl.ds((j * 16 + u) * LANES, LANES)] = z
        return 0
    lax.fori_loop(0, n // (16 * LANES), body, 0)


def _lane_reduce_i32(hist, red, nbins, stride):
    def body(j, _):
        acc = jnp.zeros((LANES,), jnp.int32)
        for l in range(LANES):
            acc = acc + hist[pl.ds(l * stride + j * LANES, LANES)]
        red[pl.ds(j * LANES, LANES)] = acc
        return 0
    lax.fori_loop(0, nbins // LANES, body, 0)


def _hist1_sc_body(nl_hbm, out_hbm, buf0, buf1, hist, red, sem0, sem1):
    wid = lax.axis_index("s") * NCORES + lax.axis_index("c")
    base = wid * ELEMS
    lane_off = lax.iota(jnp.int32, LANES) * STRIDE1
    ones = jnp.ones((LANES,), jnp.int32)

    _zero_vmem_i32(hist, HSZ1)

    def process_chunk(buf, carry):
        @plsc.parallel_loop(0, CHUNK // LANES, unroll=UN)
        def body(i):
            v = buf[pl.ds(i * LANES, LANES)]
            idx = (v * SCALE1).astype(jnp.int32)
            plsc.addupdate_scatter(hist, [lane_off + idx], ones)
        return carry

    _stream_chunks(nl_hbm, base, (buf0, buf1), (sem0, sem1),
                   process_chunk, 0)
    _lane_reduce_i32(hist, red, NB1, STRIDE1)
    pltpu.sync_copy(red, out_hbm.at[wid])


@functools.cache
def _hist1_sc():
    return pl.kernel(
        _hist1_sc_body,
        out_type=jax.ShapeDtypeStruct((NWORK, NB1), jnp.int32),
        mesh=_sc_mesh(),
        compiler_params=pltpu.CompilerParams(needs_layout_passes=False),
        scratch_types=[
            pltpu.VMEM((CHUNK,), jnp.float32),
            pltpu.VMEM((CHUNK,), jnp.float32),
            pltpu.VMEM((HSZ1,), jnp.int32),
            pltpu.VMEM((NB1,), jnp.int32),
            pltpu.SemaphoreType.DMA,
            pltpu.SemaphoreType.DMA,
        ],
    )


def _hist2_sc_body(nl_hbm, params_hbm, ocnt_hbm, oabove_hbm,
                   buf0, buf1, pbuf, hcnt, rcnt, avec, sem0, sem1):
    wid = lax.axis_index("s") * NCORES + lax.axis_index("c")
    base = wid * ELEMS
    lane_off = lax.iota(jnp.int32, LANES) * STRIDE2
    ones = jnp.ones((LANES,), jnp.int32)
    zf = jnp.zeros((LANES,), jnp.float32)

    pltpu.sync_copy(params_hbm, pbuf)
    t1i = pbuf[0].astype(jnp.int32)
    lov = pbuf[1]

    _zero_vmem_i32(hcnt, HSZ2)

    def process_chunk(buf, acc):
        def body(i, acc):
            v = buf[pl.ds(i * LANES, LANES)]
            b1 = (v * SCALE1).astype(jnp.int32)
            gt = (b1 > t1i) & (b1 < NB1)
            eq = b1 == t1i
            b2 = jnp.minimum(jnp.maximum(
                ((v - lov) * SCALE2).astype(jnp.int32), 0), NB2 - 1)
            plsc.addupdate_scatter(hcnt, [lane_off + b2], ones, mask=eq)
            return acc + jnp.where(gt, v, 0.0)
        return plsc.parallel_loop(0, CHUNK // LANES, unroll=UN,
                                  carry=acc)(body)

    acc = _stream_chunks(nl_hbm, base, (buf0, buf1), (sem0, sem1),
                         process_chunk, zf)
    avec[...] = acc
    _lane_reduce_i32(hcnt, rcnt, NB2, STRIDE2)
    pltpu.sync_copy(rcnt, ocnt_hbm.at[wid])
    pltpu.sync_copy(avec, oabove_hbm.at[wid])


@functools.cache
def _hist2_sc():
    return pl.kernel(
        _hist2_sc_body,
        out_type=(
            jax.ShapeDtypeStruct((NWORK, NB2), jnp.int32),
            jax.ShapeDtypeStruct((NWORK, LANES), jnp.float32),
        ),
        mesh=_sc_mesh(),
        compiler_params=pltpu.CompilerParams(needs_layout_passes=False),
        scratch_types=[
            pltpu.VMEM((CHUNK,), jnp.float32),
            pltpu.VMEM((CHUNK,), jnp.float32),
            pltpu.VMEM((8, LANES), jnp.float32),
            pltpu.VMEM((HSZ2,), jnp.int32),
            pltpu.VMEM((NB2,), jnp.int32),
            pltpu.VMEM((LANES,), jnp.float32),
            pltpu.SemaphoreType.DMA,
            pltpu.SemaphoreType.DMA,
        ],
    )


def kernel(pred, shrink_map, shrink_mask, threshold_map, threshold_mask):
    sums, negloss = _pass_a(pred, shrink_map, shrink_mask,
                            threshold_map, threshold_mask)
    s = jnp.sum(sums[:, 0, :9], axis=0)
    pos_loss_sum, pos_count, neg_count = s[0], s[1], s[2]
    neg_sum_total, l1_num, l1_den = s[3], s[4], s[5]
    inter, dice_a, dice_b = s[6], s[7], s[8]

    k = jnp.minimum(neg_count, pos_count * OHEM_RATIO)

    nl = negloss.reshape(-1)

    cnt1 = _hist1_sc()(nl).sum(axis=0)
    ssum1 = jnp.concatenate(
        [jnp.cumsum(cnt1[::-1])[::-1].astype(jnp.float32), jnp.zeros((1,), jnp.float32)])
    bins1 = jnp.arange(NB1)
    t1 = jnp.max(jnp.where(ssum1[:NB1] > k, bins1, -1))
    c_above1 = ssum1[t1 + 1]
    r1 = k - c_above1
    lo = t1.astype(jnp.float32) * W1

    params = jnp.stack([
        jnp.full((LANES,), t1.astype(jnp.float32)),
        jnp.full((LANES,), lo),
    ] + [jnp.zeros((LANES,), jnp.float32)] * 6)
    cnt2w, abovew = _hist2_sc()(nl, params)
    cnt2 = cnt2w.sum(axis=0)
    sum_above1 = abovew.sum()
    mids = lo + (jnp.arange(NB2, dtype=jnp.float32) + 0.5) * (W1 / NB2)
    cnt2f = cnt2.astype(jnp.float32)
    ssum2 = jnp.concatenate(
        [jnp.cumsum(cnt2f[::-1])[::-1], jnp.zeros((1,), jnp.float32)])
    vsum2 = jnp.concatenate(
        [jnp.cumsum((cnt2f * mids)[::-1])[::-1], jnp.zeros((1,), jnp.float32)])
    bins2 = jnp.arange(NB2)
    t2 = jnp.max(jnp.where(ssum2[:NB2] > r1, bins2, -1))
    c_above2 = ssum2[t2 + 1]
    sum_above2 = vsum2[t2 + 1]
    r2 = r1 - c_above2
    avg_t2 = jnp.where(t2 >= 0, mids[t2], 0.0)
    s_sel = sum_above1 + sum_above2 + r2 * avg_t2

    topk_sum = jnp.where(k >= neg_count, neg_sum_total, s_sel)

    loss_shrink = (pos_loss_sum + topk_sum) / (pos_count + k + EPS)
    loss_threshold = l1_num / (l1_den + EPS)
    loss_binary = 1.0 - 2.0 * inter / (dice_a + dice_b + EPS)
    loss_all = ALPHA * loss_shrink + BETA * loss_threshold + loss_binary
    return (loss_all, loss_shrink, loss_threshold, loss_binary)


# speedup vs baseline: 33.6688x; 1.1214x over previous
import functools

import jax
import jax.numpy as jnp
from jax import lax
from jax.experimental import pallas as pl
from jax.experimental.pallas import tpu as pltpu
from jax.experimental.pallas import tpu_sc as plsc

ALPHA = 1.0
BETA = 10.0
OHEM_RATIO = 3.0
EPS = 1e-06

N, C, H, W = 16, 3, 512, 512
TOTAL = N * H * W

VMAX = 16.2
NB1 = 4096
NB2 = 4096
SCALE1 = NB1 / VMAX
W1 = VMAX / NB1
SCALE2 = NB2 / W1
SENTINEL = 16.202


def _pass_a_body(pred_ref, smap_ref, smask_ref, tmap_ref, tmask_ref,
                 sums_ref, negloss_ref):
    sm = pred_ref[0, 0]
    tm = pred_ref[0, 1]
    bm = pred_ref[0, 2]
    y = smap_ref[0]
    m = smask_ref[0]
    t = tmap_ref[0]
    tmk = tmask_ref[0]

    pos = y * m
    neg = (1.0 - y) * m
    p = jnp.clip(sm, 1e-7, 1.0 - 1e-7)
    bce = -jnp.log(jnp.where(y > 0.5, p, 1.0 - p))

    pos_loss = bce * pos
    neg_loss = bce * neg
    negloss_ref[0] = jnp.where(neg > 0.5, neg_loss, SENTINEL)

    s0 = jnp.sum(pos_loss)
    s1 = jnp.sum(pos)
    s2 = jnp.sum(neg)
    s3 = jnp.sum(neg_loss)
    s4 = jnp.sum(jnp.abs(tm - t) * tmk)
    s5 = jnp.sum(tmk)
    s6 = jnp.sum(bm * y * m)
    s7 = jnp.sum(bm * m)
    s8 = jnp.sum(y * m)

    lane = lax.broadcasted_iota(jnp.int32, (1, 1, 128), 2)
    vec = jnp.zeros((1, 1, 128), jnp.float32)
    for j, s in enumerate((s0, s1, s2, s3, s4, s5, s6, s7, s8)):
        vec = jnp.where(lane == j, s, vec)
    sums_ref[...] = vec


def _pass_a(pred, smap, smask, tmap, tmask):
    return pl.pallas_call(
        _pass_a_body,
        grid=(N,),
        in_specs=[
            pl.BlockSpec((1, C, H, W), lambda i: (i, 0, 0, 0)),
            pl.BlockSpec((1, H, W), lambda i: (i, 0, 0)),
            pl.BlockSpec((1, H, W), lambda i: (i, 0, 0)),
            pl.BlockSpec((1, H, W), lambda i: (i, 0, 0)),
            pl.BlockSpec((1, H, W), lambda i: (i, 0, 0)),
        ],
        out_specs=[
            pl.BlockSpec((1, 1, 128), lambda i: (i, 0, 0)),
            pl.BlockSpec((1, H, W), lambda i: (i, 0, 0)),
        ],
        out_shape=[
            jax.ShapeDtypeStruct((N, 1, 128), jnp.float32),
            jax.ShapeDtypeStruct((N, H, W), jnp.float32),
        ],
    )(pred, smap, smask, tmap, tmask)


NCORES = 2
NSUB = 16
NWORK = NCORES * NSUB
ELEMS = TOTAL // NWORK
CHUNK = 16384
NCH = ELEMS // CHUNK
LANES = 16
UN = 8
STRIDE1 = NB1 + 1
STRIDE2 = NB2 + 1
HSZ1 = -(-LANES * STRIDE1 // (LANES * 16)) * (LANES * 16)
HSZ2 = -(-LANES * STRIDE2 // (LANES * 16)) * (LANES * 16)

@functools.cache
def _sc_mesh():
    return plsc.VectorSubcoreMesh(core_axis_name="c", subcore_axis_name="s",
                                  num_cores=NCORES, num_subcores=NSUB)


CROWS = CHUNK // W


def _stream_chunks(nl_hbm, wid, bufs, sems, process_chunk, carry):
    n = wid // 2
    rbase = (wid % 2) * (H // 2)

    def src(c):
        return nl_hbm.at[n, pl.ds(rbase + c * CROWS, CROWS), :]

    handles = [None, None]
    handles[0] = pltpu.async_copy(src(0), bufs[0], sems[0])
    for c in range(NCH):
        if c + 1 < NCH:
            handles[(c + 1) % 2] = pltpu.async_copy(
                src(c + 1), bufs[(c + 1) % 2], sems[(c + 1) % 2])
        handles[c % 2].wait()
        carry = process_chunk(bufs[c % 2], carry)
    return carry


def _zero_vmem_i32(ref, n):
    z = jnp.zeros((LANES,), jnp.int32)

    def body(j, _):
        for u in range(16):
            ref[pl.ds((j * 16 + u) * LANES, LANES)] = z
        return 0
    lax.fori_loop(0, n // (16 * LANES), body, 0)


def _lane_reduce_i32(hist, red, nbins, stride):
    def body(j, _):
        acc = jnp.zeros((LANES,), jnp.int32)
        for l in range(LANES):
            acc = acc + hist[pl.ds(l * stride + j * LANES, LANES)]
        red[pl.ds(j * LANES, LANES)] = acc
        return 0
    lax.fori_loop(0, nbins // LANES, body, 0)


def _hist1_sc_body(nl_hbm, out_hbm, buf0, buf1, hist, red, sem0, sem1):
    wid = lax.axis_index("s") * NCORES + lax.axis_index("c")
    lane_off = lax.iota(jnp.int32, LANES) * STRIDE1
    ones = jnp.ones((LANES,), jnp.int32)

    _zero_vmem_i32(hist, HSZ1)

    def process_chunk(buf, carry):
        @plsc.parallel_loop(0, CHUNK // LANES, unroll=UN)
        def body(i):
            r = i >> 5
            c = (i & 31) * LANES
            v = buf[r, pl.ds(c, LANES)]
            idx = (v * SCALE1).astype(jnp.int32)
            plsc.addupdate_scatter(hist, [lane_off + idx], ones)
        return carry

    _stream_chunks(nl_hbm, wid, (buf0, buf1), (sem0, sem1),
                   process_chunk, 0)
    _lane_reduce_i32(hist, red, NB1, STRIDE1)
    pltpu.sync_copy(red, out_hbm.at[wid])


@functools.cache
def _hist1_sc():
    return pl.kernel(
        _hist1_sc_body,
        out_type=jax.ShapeDtypeStruct((NWORK, NB1), jnp.int32),
        mesh=_sc_mesh(),
        compiler_params=pltpu.CompilerParams(needs_layout_passes=False),
        scratch_types=[
            pltpu.VMEM((CROWS, W), jnp.float32),
            pltpu.VMEM((CROWS, W), jnp.float32),
            pltpu.VMEM((HSZ1,), jnp.int32),
            pltpu.VMEM((NB1,), jnp.int32),
            pltpu.SemaphoreType.DMA,
            pltpu.SemaphoreType.DMA,
        ],
    )


def _hist2_sc_body(nl_hbm, params_hbm, ocnt_hbm, oabove_hbm,
                   buf0, buf1, pbuf, hcnt, rcnt, avec, sem0, sem1):
    wid = lax.axis_index("s") * NCORES + lax.axis_index("c")
    lane_off = lax.iota(jnp.int32, LANES) * STRIDE2
    ones = jnp.ones((LANES,), jnp.int32)
    zf = jnp.zeros((LANES,), jnp.float32)

    pltpu.sync_copy(params_hbm, pbuf)
    t1i = pbuf[0].astype(jnp.int32)
    lov = pbuf[1]

    _zero_vmem_i32(hcnt, HSZ2)

    def process_chunk(buf, acc):
        def body(i, acc):
            r = i >> 5
            c = (i & 31) * LANES
            v = buf[r, pl.ds(c, LANES)]
            b1 = (v * SCALE1).astype(jnp.int32)
            gt = (b1 > t1i) & (b1 < NB1)
            eq = b1 == t1i
            b2 = jnp.minimum(jnp.maximum(
                ((v - lov) * SCALE2).astype(jnp.int32), 0), NB2 - 1)
            plsc.addupdate_scatter(hcnt, [lane_off + b2], ones, mask=eq)
            return acc + jnp.where(gt, v, 0.0)
        return plsc.parallel_loop(0, CHUNK // LANES, unroll=UN,
                                  carry=acc)(body)

    acc = _stream_chunks(nl_hbm, wid, (buf0, buf1), (sem0, sem1),
                         process_chunk, zf)
    avec[...] = acc
    _lane_reduce_i32(hcnt, rcnt, NB2, STRIDE2)
    pltpu.sync_copy(rcnt, ocnt_hbm.at[wid])
    pltpu.sync_copy(avec, oabove_hbm.at[wid])


@functools.cache
def _hist2_sc():
    return pl.kernel(
        _hist2_sc_body,
        out_type=(
            jax.ShapeDtypeStruct((NWORK, NB2), jnp.int32),
            jax.ShapeDtypeStruct((NWORK, LANES), jnp.float32),
        ),
        mesh=_sc_mesh(),
        compiler_params=pltpu.CompilerParams(needs_layout_passes=False),
        scratch_types=[
            pltpu.VMEM((CROWS, W), jnp.float32),
            pltpu.VMEM((CROWS, W), jnp.float32),
            pltpu.VMEM((8, LANES), jnp.float32),
            pltpu.VMEM((HSZ2,), jnp.int32),
            pltpu.VMEM((NB2,), jnp.int32),
            pltpu.VMEM((LANES,), jnp.float32),
            pltpu.SemaphoreType.DMA,
            pltpu.SemaphoreType.DMA,
        ],
    )


def kernel(pred, shrink_map, shrink_mask, threshold_map, threshold_mask):
    sums, negloss = _pass_a(pred, shrink_map, shrink_mask,
                            threshold_map, threshold_mask)
    s = jnp.sum(sums[:, 0, :9], axis=0)
    pos_loss_sum, pos_count, neg_count = s[0], s[1], s[2]
    neg_sum_total, l1_num, l1_den = s[3], s[4], s[5]
    inter, dice_a, dice_b = s[6], s[7], s[8]

    k = jnp.minimum(neg_count, pos_count * OHEM_RATIO)

    cnt1 = _hist1_sc()(negloss).sum(axis=0)
    ssum1 = jnp.concatenate(
        [jnp.cumsum(cnt1[::-1])[::-1].astype(jnp.float32), jnp.zeros((1,), jnp.float32)])
    bins1 = jnp.arange(NB1)
    t1 = jnp.max(jnp.where(ssum1[:NB1] > k, bins1, -1))
    c_above1 = ssum1[t1 + 1]
    r1 = k - c_above1
    lo = t1.astype(jnp.float32) * W1

    params = jnp.stack([
        jnp.full((LANES,), t1.astype(jnp.float32)),
        jnp.full((LANES,), lo),
    ] + [jnp.zeros((LANES,), jnp.float32)] * 6)
    cnt2w, abovew = _hist2_sc()(negloss, params)
    cnt2 = cnt2w.sum(axis=0)
    sum_above1 = abovew.sum()
    mids = lo + (jnp.arange(NB2, dtype=jnp.float32) + 0.5) * (W1 / NB2)
    cnt2f = cnt2.astype(jnp.float32)
    ssum2 = jnp.concatenate(
        [jnp.cumsum(cnt2f[::-1])[::-1], jnp.zeros((1,), jnp.float32)])
    vsum2 = jnp.concatenate(
        [jnp.cumsum((cnt2f * mids)[::-1])[::-1], jnp.zeros((1,), jnp.float32)])
    bins2 = jnp.arange(NB2)
    t2 = jnp.max(jnp.where(ssum2[:NB2] > r1, bins2, -1))
    c_above2 = ssum2[t2 + 1]
    sum_above2 = vsum2[t2 + 1]
    r2 = r1 - c_above2
    avg_t2 = jnp.where(t2 >= 0, mids[t2], 0.0)
    s_sel = sum_above1 + sum_above2 + r2 * avg_t2

    topk_sum = jnp.where(k >= neg_count, neg_sum_total, s_sel)

    loss_shrink = (pos_loss_sum + topk_sum) / (pos_count + k + EPS)
    loss_threshold = l1_num / (l1_den + EPS)
    loss_binary = 1.0 - 2.0 * inter / (dice_a + dice_b + EPS)
    loss_all = ALPHA * loss_shrink + BETA * loss_threshold + loss_binary
    return (loss_all, loss_shrink, loss_threshold, loss_binary)


# speedup vs baseline: 36.1703x; 1.0743x over previous
import functools

import jax
import jax.numpy as jnp
from jax import lax
from jax.experimental import pallas as pl
from jax.experimental.pallas import tpu as pltpu
from jax.experimental.pallas import tpu_sc as plsc

ALPHA = 1.0
BETA = 10.0
OHEM_RATIO = 3.0
EPS = 1e-06

N, C, H, W = 16, 3, 512, 512
TOTAL = N * H * W

VMAX = 16.2
NB1 = 4096
NB2 = 4096
SCALE1 = NB1 / VMAX
W1 = VMAX / NB1
SCALE2 = NB2 / W1
LOG2NB2 = 12
BIGSCALE = NB1 * NB2 / VMAX
SENTINEL = 16.202


def _pass_a_body(pred_ref, smap_ref, smask_ref, tmap_ref, tmask_ref,
                 sums_ref, negloss_ref):
    sm = pred_ref[0, 0]
    tm = pred_ref[0, 1]
    bm = pred_ref[0, 2]
    y = smap_ref[0]
    m = smask_ref[0]
    t = tmap_ref[0]
    tmk = tmask_ref[0]

    pos = y * m
    neg = (1.0 - y) * m
    p = jnp.clip(sm, 1e-7, 1.0 - 1e-7)
    bce = -jnp.log(jnp.where(y > 0.5, p, 1.0 - p))

    pos_loss = bce * pos
    neg_loss = bce * neg
    negloss_ref[0] = jnp.where(neg > 0.5, neg_loss, SENTINEL)

    s0 = jnp.sum(pos_loss)
    s1 = jnp.sum(pos)
    s2 = jnp.sum(neg)
    s3 = jnp.sum(neg_loss)
    s4 = jnp.sum(jnp.abs(tm - t) * tmk)
    s5 = jnp.sum(tmk)
    s6 = jnp.sum(bm * y * m)
    s7 = jnp.sum(bm * m)
    s8 = jnp.sum(y * m)

    lane = lax.broadcasted_iota(jnp.int32, (1, 1, 128), 2)
    vec = jnp.zeros((1, 1, 128), jnp.float32)
    for j, s in enumerate((s0, s1, s2, s3, s4, s5, s6, s7, s8)):
        vec = jnp.where(lane == j, s, vec)
    sums_ref[...] = vec


def _pass_a(pred, smap, smask, tmap, tmask):
    return pl.pallas_call(
        _pass_a_body,
        grid=(N,),
        in_specs=[
            pl.BlockSpec((1, C, H, W), lambda i: (i, 0, 0, 0)),
            pl.BlockSpec((1, H, W), lambda i: (i, 0, 0)),
            pl.BlockSpec((1, H, W), lambda i: (i, 0, 0)),
            pl.BlockSpec((1, H, W), lambda i: (i, 0, 0)),
            pl.BlockSpec((1, H, W), lambda i: (i, 0, 0)),
        ],
        out_specs=[
            pl.BlockSpec((1, 1, 128), lambda i: (i, 0, 0)),
            pl.BlockSpec((1, H, W), lambda i: (i, 0, 0)),
        ],
        out_shape=[
            jax.ShapeDtypeStruct((N, 1, 128), jnp.float32),
            jax.ShapeDtypeStruct((N, H, W), jnp.float32),
        ],
    )(pred, smap, smask, tmap, tmask)


NCORES = 2
NSUB = 16
NWORK = NCORES * NSUB
ELEMS = TOTAL // NWORK
CHUNK = 16384
NCH = ELEMS // CHUNK
LANES = 16
UN = 8
STRIDE1 = NB1 + 1
STRIDE2 = NB2 + 1
HSZ1 = -(-LANES * STRIDE1 // (LANES * 16)) * (LANES * 16)
HSZ2 = -(-LANES * STRIDE2 // (LANES * 16)) * (LANES * 16)

@functools.cache
def _sc_mesh():
    return plsc.VectorSubcoreMesh(core_axis_name="c", subcore_axis_name="s",
                                  num_cores=NCORES, num_subcores=NSUB)


CROWS = CHUNK // W


def _stream_chunks(nl_hbm, wid, bufs, sems, process_chunk, carry):
    n = wid // 2
    rbase = (wid % 2) * (H // 2)

    def src(c):
        return nl_hbm.at[n, pl.ds(rbase + c * CROWS, CROWS), :]

    handles = [None, None]
    handles[0] = pltpu.async_copy(src(0), bufs[0], sems[0])
    for c in range(NCH):
        if c + 1 < NCH:
            handles[(c + 1) % 2] = pltpu.async_copy(
                src(c + 1), bufs[(c + 1) % 2], sems[(c + 1) % 2])
        handles[c % 2].wait()
        carry = process_chunk(bufs[c % 2], carry)
    return carry


def _zero_vmem_i32(ref, n):
    z = jnp.zeros((LANES,), jnp.int32)

    def body(j, _):
        for u in range(16):
            ref[pl.ds((j * 16 + u) * LANES, LANES)] = z
        return 0
    lax.fori_loop(0, n // (16 * LANES), body, 0)


def _lane_reduce_i32(hist, red, nbins, stride):
    def body(j, _):
        acc = jnp.zeros((LANES,), jnp.int32)
        for l in range(LANES):
            acc = acc + hist[pl.ds(l * stride + j * LANES, LANES)]
        red[pl.ds(j * LANES, LANES)] = acc
        return 0
    lax.fori_loop(0, nbins // LANES, body, 0)


def _hist1_sc_body(nl_hbm, out_hbm, buf0, buf1, hist, red, sem0, sem1):
    wid = lax.axis_index("s") * NCORES + lax.axis_index("c")
    lane_off = lax.iota(jnp.int32, LANES) * STRIDE1
    ones = jnp.ones((LANES,), jnp.int32)

    _zero_vmem_i32(hist, HSZ1)

    def process_chunk(buf, carry):
        @plsc.parallel_loop(0, CHUNK // LANES, unroll=UN)
        def body(i):
            r = i >> 5
            c = (i & 31) * LANES
            v = buf[r, pl.ds(c, LANES)]
            idx = (v * BIGSCALE).astype(jnp.int32) >> LOG2NB2
            plsc.addupdate_scatter(hist, [lane_off + idx], ones)
        return carry

    _stream_chunks(nl_hbm, wid, (buf0, buf1), (sem0, sem1),
                   process_chunk, 0)
    _lane_reduce_i32(hist, red, NB1, STRIDE1)
    pltpu.sync_copy(red, out_hbm.at[wid])


@functools.cache
def _hist1_sc():
    return pl.kernel(
        _hist1_sc_body,
        out_type=jax.ShapeDtypeStruct((NWORK, NB1), jnp.int32),
        mesh=_sc_mesh(),
        compiler_params=pltpu.CompilerParams(needs_layout_passes=False),
        scratch_types=[
            pltpu.VMEM((CROWS, W), jnp.float32),
            pltpu.VMEM((CROWS, W), jnp.float32),
            pltpu.VMEM((HSZ1,), jnp.int32),
            pltpu.VMEM((NB1,), jnp.int32),
            pltpu.SemaphoreType.DMA,
            pltpu.SemaphoreType.DMA,
        ],
    )


def _hist2_sc_body(nl_hbm, params_hbm, ocnt_hbm, oabove_hbm,
                   buf0, buf1, pbuf, hcnt, rcnt, avec, sem0, sem1):
    wid = lax.axis_index("s") * NCORES + lax.axis_index("c")
    lane_off = lax.iota(jnp.int32, LANES) * STRIDE2
    ones = jnp.ones((LANES,), jnp.int32)
    zf = jnp.zeros((LANES,), jnp.float32)

    pltpu.sync_copy(params_hbm, pbuf)
    t1i = pbuf[0].astype(jnp.int32)
    off2 = t1i * NB2

    _zero_vmem_i32(hcnt, HSZ2)

    def process_chunk(buf, acc):
        def body(i, acc):
            r = i >> 5
            c = (i & 31) * LANES
            v = buf[r, pl.ds(c, LANES)]
            f = (v * BIGSCALE).astype(jnp.int32)
            b1 = f >> LOG2NB2
            gt = (b1 > t1i) & (b1 < NB1)
            eq = b1 == t1i
            plsc.addupdate_scatter(hcnt, [lane_off + (f - off2)], ones,
                                   mask=eq)
            return acc + jnp.where(gt, v, 0.0)
        return plsc.parallel_loop(0, CHUNK // LANES, unroll=UN,
                                  carry=acc)(body)

    acc = _stream_chunks(nl_hbm, wid, (buf0, buf1), (sem0, sem1),
                         process_chunk, zf)
    avec[...] = acc
    _lane_reduce_i32(hcnt, rcnt, NB2, STRIDE2)
    pltpu.sync_copy(rcnt, ocnt_hbm.at[wid])
    pltpu.sync_copy(avec, oabove_hbm.at[wid])


@functools.cache
def _hist2_sc():
    return pl.kernel(
        _hist2_sc_body,
        out_type=(
            jax.ShapeDtypeStruct((NWORK, NB2), jnp.int32),
            jax.ShapeDtypeStruct((NWORK, LANES), jnp.float32),
        ),
        mesh=_sc_mesh(),
        compiler_params=pltpu.CompilerParams(needs_layout_passes=False),
        scratch_types=[
            pltpu.VMEM((CROWS, W), jnp.float32),
            pltpu.VMEM((CROWS, W), jnp.float32),
            pltpu.VMEM((8, LANES), jnp.float32),
            pltpu.VMEM((HSZ2,), jnp.int32),
            pltpu.VMEM((NB2,), jnp.int32),
            pltpu.VMEM((LANES,), jnp.float32),
            pltpu.SemaphoreType.DMA,
            pltpu.SemaphoreType.DMA,
        ],
    )


def kernel(pred, shrink_map, shrink_mask, threshold_map, threshold_mask):
    sums, negloss = _pass_a(pred, shrink_map, shrink_mask,
                            threshold_map, threshold_mask)
    s = jnp.sum(sums[:, 0, :9], axis=0)
    pos_loss_sum, pos_count, neg_count = s[0], s[1], s[2]
    neg_sum_total, l1_num, l1_den = s[3], s[4], s[5]
    inter, dice_a, dice_b = s[6], s[7], s[8]

    k = jnp.minimum(neg_count, pos_count * OHEM_RATIO)

    cnt1 = _hist1_sc()(negloss).sum(axis=0)
    ssum1 = jnp.concatenate(
        [jnp.cumsum(cnt1[::-1])[::-1].astype(jnp.float32), jnp.zeros((1,), jnp.float32)])
    bins1 = jnp.arange(NB1)
    t1 = jnp.max(jnp.where(ssum1[:NB1] > k, bins1, -1))
    c_above1 = ssum1[t1 + 1]
    r1 = k - c_above1
    lo = t1.astype(jnp.float32) * W1

    params = jnp.stack([
        jnp.full((LANES,), t1.astype(jnp.float32)),
        jnp.full((LANES,), lo),
    ] + [jnp.zeros((LANES,), jnp.float32)] * 6)
    cnt2w, abovew = _hist2_sc()(negloss, params)
    cnt2 = cnt2w.sum(axis=0)
    sum_above1 = abovew.sum()
    mids = lo + (jnp.arange(NB2, dtype=jnp.float32) + 0.5) * (W1 / NB2)
    cnt2f = cnt2.astype(jnp.float32)
    ssum2 = jnp.concatenate(
        [jnp.cumsum(cnt2f[::-1])[::-1], jnp.zeros((1,), jnp.float32)])
    vsum2 = jnp.concatenate(
        [jnp.cumsum((cnt2f * mids)[::-1])[::-1], jnp.zeros((1,), jnp.float32)])
    bins2 = jnp.arange(NB2)
    t2 = jnp.max(jnp.where(ssum2[:NB2] > r1, bins2, -1))
    c_above2 = ssum2[t2 + 1]
    sum_above2 = vsum2[t2 + 1]
    r2 = r1 - c_above2
    avg_t2 = jnp.where(t2 >= 0, mids[t2], 0.0)
    s_sel = sum_above1 + sum_above2 + r2 * avg_t2

    topk_sum = jnp.where(k >= neg_count, neg_sum_total, s_sel)

    loss_shrink = (pos_loss_sum + topk_sum) / (pos_count + k + EPS)
    loss_threshold = l1_num / (l1_den + EPS)
    loss_binary = 1.0 - 2.0 * inter / (dice_a + dice_b + EPS)
    loss_all = ALPHA * loss_shrink + BETA * loss_threshold + loss_binary
    return (loss_all, loss_shrink, loss_threshold, loss_binary)


# speedup vs baseline: 55.9843x; 1.5478x over previous
import functools

import jax
import jax.numpy as jnp
from jax import lax
from jax.experimental import pallas as pl
from jax.experimental.pallas import tpu as pltpu
from jax.experimental.pallas import tpu_sc as plsc

ALPHA = 1.0
BETA = 10.0
OHEM_RATIO = 3.0
EPS = 1e-06

N, C, H, W = 16, 3, 512, 512
TOTAL = N * H * W

VMAX = 16.2
NB1 = 4096
NB2 = 4096
SCALE1 = NB1 / VMAX
W1 = VMAX / NB1
SCALE2 = NB2 / W1
LOG2NB2 = 12
BIGSCALE = NB1 * NB2 / VMAX
SENTINEL = 16.202


def _pass_a_body(pred_ref, smap_ref, smask_ref, tmap_ref, tmask_ref,
                 sums_ref, negloss_ref):
    sm = pred_ref[0, 0]
    tm = pred_ref[0, 1]
    bm = pred_ref[0, 2]
    y = smap_ref[0]
    m = smask_ref[0]
    t = tmap_ref[0]
    tmk = tmask_ref[0]

    pos = y * m
    neg = (1.0 - y) * m
    p = jnp.clip(sm, 1e-7, 1.0 - 1e-7)
    bce = -jnp.log(jnp.where(y > 0.5, p, 1.0 - p))

    pos_loss = bce * pos
    neg_loss = bce * neg
    negloss_ref[0] = jnp.where(neg > 0.5, neg_loss, SENTINEL)

    s0 = jnp.sum(pos_loss)
    s1 = jnp.sum(pos)
    s2 = jnp.sum(neg)
    s3 = jnp.sum(neg_loss)
    s4 = jnp.sum(jnp.abs(tm - t) * tmk)
    s5 = jnp.sum(tmk)
    s6 = jnp.sum(bm * y * m)
    s7 = jnp.sum(bm * m)
    s8 = jnp.sum(y * m)

    lane = lax.broadcasted_iota(jnp.int32, (1, 1, 128), 2)
    vec = jnp.zeros((1, 1, 128), jnp.float32)
    for j, s in enumerate((s0, s1, s2, s3, s4, s5, s6, s7, s8)):
        vec = jnp.where(lane == j, s, vec)
    sums_ref[...] = vec


def _pass_a(pred, smap, smask, tmap, tmask):
    return pl.pallas_call(
        _pass_a_body,
        grid=(N,),
        in_specs=[
            pl.BlockSpec((1, C, H, W), lambda i: (i, 0, 0, 0)),
            pl.BlockSpec((1, H, W), lambda i: (i, 0, 0)),
            pl.BlockSpec((1, H, W), lambda i: (i, 0, 0)),
            pl.BlockSpec((1, H, W), lambda i: (i, 0, 0)),
            pl.BlockSpec((1, H, W), lambda i: (i, 0, 0)),
        ],
        out_specs=[
            pl.BlockSpec((1, 1, 128), lambda i: (i, 0, 0)),
            pl.BlockSpec((1, H, W), lambda i: (i, 0, 0)),
        ],
        out_shape=[
            jax.ShapeDtypeStruct((N, 1, 128), jnp.float32),
            jax.ShapeDtypeStruct((N, H, W), jnp.float32),
        ],
    )(pred, smap, smask, tmap, tmask)


NCORES = 2
NSUB = 16
NWORK = NCORES * NSUB
ELEMS = TOTAL // NWORK
CHUNK = 16384
NCH = ELEMS // CHUNK
LANES = 16
UN = 8
STRIDE1 = NB1 + 1
STRIDE2 = NB2 + 1
HSZ1 = -(-LANES * STRIDE1 // (LANES * 16)) * (LANES * 16)
HSZ2 = -(-LANES * STRIDE2 // (LANES * 16)) * (LANES * 16)

@functools.cache
def _sc_mesh():
    return plsc.VectorSubcoreMesh(core_axis_name="c", subcore_axis_name="s",
                                  num_cores=NCORES, num_subcores=NSUB)


CROWS = CHUNK // W


def _stream_chunks(nl_hbm, wid, bufs, sems, process_chunk, carry):
    n = wid // 2
    rbase = (wid % 2) * (H // 2)

    def src(c):
        return nl_hbm.at[n, pl.ds(rbase + c * CROWS, CROWS), :]

    handles = [None, None]
    handles[0] = pltpu.async_copy(src(0), bufs[0], sems[0])
    for c in range(NCH):
        if c + 1 < NCH:
            handles[(c + 1) % 2] = pltpu.async_copy(
                src(c + 1), bufs[(c + 1) % 2], sems[(c + 1) % 2])
        handles[c % 2].wait()
        carry = process_chunk(bufs[c % 2], carry)
    return carry


def _zero_vmem_i32(ref, n):
    z = jnp.zeros((LANES,), jnp.int32)

    def body(j, _):
        for u in range(16):
            ref[pl.ds((j * 16 + u) * LANES, LANES)] = z
        return 0
    lax.fori_loop(0, n // (16 * LANES), body, 0)


def _lane_reduce_i32(hist, red, nbins, stride):
    def body(j, _):
        acc = jnp.zeros((LANES,), jnp.int32)
        for l in range(LANES):
            acc = acc + hist[pl.ds(l * stride + j * LANES, LANES)]
        red[pl.ds(j * LANES, LANES)] = acc
        return 0
    lax.fori_loop(0, nbins // LANES, body, 0)


def _hist1_sc_body(nl_hbm, out_hbm, buf0, buf1, hist, red, sem0, sem1):
    wid = lax.axis_index("s") * NCORES + lax.axis_index("c")
    lane_off = lax.iota(jnp.int32, LANES) * STRIDE1
    ones = jnp.ones((LANES,), jnp.int32)

    _zero_vmem_i32(hist, HSZ1)

    def process_chunk(buf, carry):
        @plsc.parallel_loop(0, CHUNK // LANES, unroll=UN)
        def body(i):
            r = i >> 5
            c = (i & 31) * LANES
            v = buf[r, pl.ds(c, LANES)]
            idx = (v * BIGSCALE).astype(jnp.int32) >> LOG2NB2
            plsc.addupdate_scatter(hist, [lane_off + idx], ones)
        return carry

    _stream_chunks(nl_hbm, wid, (buf0, buf1), (sem0, sem1),
                   process_chunk, 0)
    _lane_reduce_i32(hist, red, NB1, STRIDE1)
    pltpu.sync_copy(red, out_hbm.at[wid])


@functools.cache
def _hist1_sc():
    return pl.kernel(
        _hist1_sc_body,
        out_type=jax.ShapeDtypeStruct((NWORK, NB1), jnp.int32),
        mesh=_sc_mesh(),
        compiler_params=pltpu.CompilerParams(needs_layout_passes=False),
        scratch_types=[
            pltpu.VMEM((CROWS, W), jnp.float32),
            pltpu.VMEM((CROWS, W), jnp.float32),
            pltpu.VMEM((HSZ1,), jnp.int32),
            pltpu.VMEM((NB1,), jnp.int32),
            pltpu.SemaphoreType.DMA,
            pltpu.SemaphoreType.DMA,
        ],
    )


def _hist2_sc_body(nl_hbm, params_hbm, ocnt_hbm, oabove_hbm,
                   buf0, buf1, pbuf, hcnt, rcnt, avec, sem0, sem1):
    wid = lax.axis_index("s") * NCORES + lax.axis_index("c")
    lane_off = lax.iota(jnp.int32, LANES) * STRIDE2
    ones = jnp.ones((LANES,), jnp.int32)
    zf = jnp.zeros((LANES,), jnp.float32)

    pltpu.sync_copy(params_hbm, pbuf)
    t1i = pbuf[0].astype(jnp.int32)
    off2 = t1i * NB2

    _zero_vmem_i32(hcnt, HSZ2)

    def process_chunk(buf, acc):
        def body(i, acc):
            r = i >> 5
            c = (i & 31) * LANES
            v = buf[r, pl.ds(c, LANES)]
            f = (v * BIGSCALE).astype(jnp.int32)
            b1 = f >> LOG2NB2
            gt = (b1 > t1i) & (b1 < NB1)
            eq = b1 == t1i
            plsc.addupdate_scatter(hcnt, [lane_off + (f - off2)], ones,
                                   mask=eq)
            return acc + jnp.where(gt, v, 0.0)
        return plsc.parallel_loop(0, CHUNK // LANES, unroll=UN,
                                  carry=acc)(body)

    acc = _stream_chunks(nl_hbm, wid, (buf0, buf1), (sem0, sem1),
                         process_chunk, zf)
    avec[...] = acc
    _lane_reduce_i32(hcnt, rcnt, NB2, STRIDE2)
    pltpu.sync_copy(rcnt, ocnt_hbm.at[wid])
    pltpu.sync_copy(avec, oabove_hbm.at[wid])


@functools.cache
def _hist2_sc():
    return pl.kernel(
        _hist2_sc_body,
        out_type=(
            jax.ShapeDtypeStruct((NWORK, NB2), jnp.int32),
            jax.ShapeDtypeStruct((NWORK, LANES), jnp.float32),
        ),
        mesh=_sc_mesh(),
        compiler_params=pltpu.CompilerParams(needs_layout_passes=False),
        scratch_types=[
            pltpu.VMEM((CROWS, W), jnp.float32),
            pltpu.VMEM((CROWS, W), jnp.float32),
            pltpu.VMEM((8, LANES), jnp.float32),
            pltpu.VMEM((HSZ2,), jnp.int32),
            pltpu.VMEM((NB2,), jnp.int32),
            pltpu.VMEM((LANES,), jnp.float32),
            pltpu.SemaphoreType.DMA,
            pltpu.SemaphoreType.DMA,
        ],
    )


def kernel(pred, shrink_map, shrink_mask, threshold_map, threshold_mask):
    sums, negloss = _pass_a(pred, shrink_map, shrink_mask,
                            threshold_map, threshold_mask)
    s = jnp.sum(sums[:, 0, :9], axis=0)
    pos_loss_sum, pos_count, neg_count = s[0], s[1], s[2]
    neg_sum_total, l1_num, l1_den = s[3], s[4], s[5]
    inter, dice_a, dice_b = s[6], s[7], s[8]

    k = jnp.minimum(neg_count, pos_count * OHEM_RATIO)

    cnt1w = _hist1_sc()(negloss)

    def _take_all(ops):
        return ops[3]

    def _selection_sum(ops):
        cnt1w, negloss, k, _ = ops
        cnt1 = cnt1w.sum(axis=0)
        ssum1 = jnp.concatenate(
            [jnp.cumsum(cnt1[::-1])[::-1].astype(jnp.float32),
             jnp.zeros((1,), jnp.float32)])
        bins1 = jnp.arange(NB1)
        t1 = jnp.max(jnp.where(ssum1[:NB1] > k, bins1, -1))
        c_above1 = ssum1[t1 + 1]
        r1 = k - c_above1
        lo = t1.astype(jnp.float32) * W1

        params = jnp.stack([
            jnp.full((LANES,), t1.astype(jnp.float32)),
            jnp.full((LANES,), lo),
        ] + [jnp.zeros((LANES,), jnp.float32)] * 6)
        cnt2w, abovew = _hist2_sc()(negloss, params)
        cnt2 = cnt2w.sum(axis=0)
        sum_above1 = abovew.sum()
        mids = lo + (jnp.arange(NB2, dtype=jnp.float32) + 0.5) * (W1 / NB2)
        cnt2f = cnt2.astype(jnp.float32)
        ssum2 = jnp.concatenate(
            [jnp.cumsum(cnt2f[::-1])[::-1], jnp.zeros((1,), jnp.float32)])
        vsum2 = jnp.concatenate(
            [jnp.cumsum((cnt2f * mids)[::-1])[::-1],
             jnp.zeros((1,), jnp.float32)])
        bins2 = jnp.arange(NB2)
        t2 = jnp.max(jnp.where(ssum2[:NB2] > r1, bins2, -1))
        c_above2 = ssum2[t2 + 1]
        sum_above2 = vsum2[t2 + 1]
        r2 = r1 - c_above2
        avg_t2 = jnp.where(t2 >= 0, mids[t2], 0.0)
        return sum_above1 + sum_above2 + r2 * avg_t2

    topk_sum = lax.cond(k >= neg_count, _take_all, _selection_sum,
                        (cnt1w, negloss, k, neg_sum_total))

    loss_shrink = (pos_loss_sum + topk_sum) / (pos_count + k + EPS)
    loss_threshold = l1_num / (l1_den + EPS)
    loss_binary = 1.0 - 2.0 * inter / (dice_a + dice_b + EPS)
    loss_all = ALPHA * loss_shrink + BETA * loss_threshold + loss_binary
    return (loss_all, loss_shrink, loss_threshold, loss_binary)
